# Initial kernel scaffold; baseline (speedup 1.0000x reference)
import jax
import jax.numpy as jnp
from jax.experimental import pallas as pl


def kernel(x, edge_index, edge_weight, W1, b1, W2, b2, W3, b3, gn1_w, gn1_b, gn1_ms, gn2_w, gn2_b, gn2_ms, m1_keys, m1_headw, m1_lin, m2_keys, m2_headw, m2_lin):
    raise NotImplementedError("write your pallas kernel here")


# speedup vs baseline: 3.1545x; 3.1545x over previous
import functools

import numpy as np
import jax
import jax.numpy as jnp
from jax import lax
from jax.experimental import pallas as pl
from jax.experimental.pallas import tpu as pltpu
from jax.experimental.pallas import tpu_sc as plsc

N = 10000
D = 128
E = 160000
K_HOPS = 3
HEADS = 3
CLUSTERS = 3

NC = 2
NS = 16
NW = NC * NS
ET = 5120
EP = NW * ET
C = 128
NCHUNK = ET // C
RPT = N // NS
NP = 10240
RPTP = NP // NS

BN = 400
NBLK = N // BN
HK = 16
CP = 8

@functools.cache
def _sc_mesh():
    return plsc.VectorSubcoreMesh(core_axis_name="c", subcore_axis_name="s",
                                  num_cores=NC, num_subcores=NS)

_prec = lax.Precision.HIGHEST


def _bdot(a, b):
    return jnp.dot(a.astype(jnp.bfloat16), b.astype(jnp.bfloat16),
                   preferred_element_type=jnp.float32)


def _f32(x):
    return jnp.full((16,), x, dtype=jnp.float32)


def _sc_norm_body(row_h, col_h, w_h, norm_h,
                  rowv, colv, wv, normv, degl, dsl, cbuf, dinvl, stage, dinvs):
    cid = lax.axis_index("c")
    sid = lax.axis_index("s")
    wid = sid * NC + cid

    def _z(i, _):
        degl[pl.ds(i * 16, 16)] = jnp.zeros((16,), jnp.float32)
        return 0
    lax.fori_loop(0, NP // 16, _z, 0)

    for r in range(2):
        b = sid * 2 + r
        pltpu.sync_copy(col_h.at[b], colv)
        pltpu.sync_copy(w_h.at[b], wv)
        for j in range(NCHUNK):
            def _deg(k, _, j=j):
                sl = pl.ds(k * 16, 16)
                plsc.addupdate_scatter(degl, [colv[j, sl]], wv[j, sl])
                return 0
            lax.fori_loop(0, C // 16, _deg, 0)

    pltpu.sync_copy(degl, stage.at[sid])
    plsc.subcore_barrier()
    pltpu.sync_copy(stage.at[:, pl.ds(sid * RPTP, RPTP)], cbuf)

    def _sum(k, _):
        sl = pl.ds(k * 16, 16)
        v = cbuf[0, sl]
        for t in range(1, NS):
            v = v + cbuf[t, sl]
        dsl[sl] = v
        return 0
    lax.fori_loop(0, RPTP // 16, _sum, 0)

    def _dinv(k, _):
        sl = pl.ds(k * 16, 16)
        d = dsl[sl]
        i = plsc.bitcast(d, jnp.int32)
        i = jnp.full((16,), 0x5F3759DF, jnp.int32) - lax.shift_right_logical(
            i, jnp.full((16,), 1, jnp.int32))
        y = plsc.bitcast(i, jnp.float32)
        for _n in range(3):
            y = y * (_f32(1.5) - _f32(0.5) * d * y * y)
        dsl[sl] = jnp.where(d > jnp.zeros((16,), jnp.float32), y,
                            jnp.zeros((16,), jnp.float32))
        return 0
    lax.fori_loop(0, RPTP // 16, _dinv, 0)
    pltpu.sync_copy(dsl, dinvs.at[pl.ds(sid * RPTP, RPTP)])
    plsc.subcore_barrier()

    pltpu.sync_copy(dinvs, dinvl)
    pltpu.sync_copy(row_h.at[wid], rowv)
    pltpu.sync_copy(col_h.at[wid], colv)
    pltpu.sync_copy(w_h.at[wid], wv)
    for j in range(NCHUNK):
        def _norm(k, _, j=j):
            sl = pl.ds(k * 16, 16)
            dr = plsc.load_gather(dinvl, [rowv[j, sl]])
            dc = plsc.load_gather(dinvl, [colv[j, sl]])
            normv[j, sl] = dr * wv[j, sl] * dc
            return 0
        lax.fori_loop(0, C // 16, _norm, 0)
    pltpu.sync_copy(normv, norm_h.at[wid])


@functools.cache
def _sc_norm_kernel():
    return pl.kernel(
        _sc_norm_body,
        out_type=jax.ShapeDtypeStruct((NW, NCHUNK, C), jnp.float32),
        mesh=_sc_mesh(),
        compiler_params=pltpu.CompilerParams(needs_layout_passes=False),
        scratch_types=[
        pltpu.VMEM((NCHUNK, C), jnp.int32),
        pltpu.VMEM((NCHUNK, C), jnp.int32),
        pltpu.VMEM((NCHUNK, C), jnp.float32),
        pltpu.VMEM((NCHUNK, C), jnp.float32),
        pltpu.VMEM((NP,), jnp.float32),
        pltpu.VMEM((RPTP,), jnp.float32),
        pltpu.VMEM((NS, RPTP), jnp.float32),
        pltpu.VMEM((NP,), jnp.float32),
        pltpu.VMEM_SHARED((NS, NP), jnp.float32),
        pltpu.VMEM_SHARED((NP,), jnp.float32),
        ],
        name="sc_edge_norm",
    )


def _sc_norm(row3, col3, w3):
    return _sc_norm_kernel()(row3, col3, w3)


def _sc_hop_body(h_h, row_h, col_h, norm_h, part_h,
                 rowv, colv, normv, gbuf, acc, sem):
    cid = lax.axis_index("c")
    sid = lax.axis_index("s")
    wid = sid * NC + cid

    pltpu.sync_copy(row_h.at[wid], rowv)
    pltpu.sync_copy(col_h.at[wid], colv)
    pltpu.sync_copy(norm_h.at[wid], normv)

    def _zg(i, _):
        for k in range(D // 16):
            gbuf[i, pl.ds(k * 16, 16)] = jnp.zeros((16,), jnp.float32)
        return 0
    lax.fori_loop(0, C, _zg, 0)
    for t in range(RPTP // C):
        pltpu.sync_copy(gbuf, acc.at[pl.ds(sid * RPTP + t * C, C)])
    plsc.subcore_barrier()

    def _chunk(j, _):
        pltpu.async_copy(h_h.at[rowv.at[j]], gbuf, sem).wait()

        def _scale(g, _):
            nv = normv[j, pl.ds(g * 16, 16)]
            for l in range(16):
                s = nv[l]
                e = g * 16 + l
                for k in range(D // 16):
                    sl = pl.ds(k * 16, 16)
                    gbuf[e, sl] = gbuf[e, sl] * s
            return 0
        lax.fori_loop(0, C // 16, _scale, 0)
        pltpu.sync_copy(gbuf, acc.at[colv.at[j]], add=True)
        return 0
    lax.fori_loop(0, NCHUNK, _chunk, 0)
    plsc.subcore_barrier()

    pltpu.sync_copy(acc.at[pl.ds(sid * RPTP, RPTP)],
                    part_h.at[cid, pl.ds(sid * RPTP, RPTP)])


@functools.cache
def _sc_hop_kernel():
    return pl.kernel(
        _sc_hop_body,
        out_type=jax.ShapeDtypeStruct((NC, NP, D), jnp.float32),
        mesh=_sc_mesh(),
        compiler_params=pltpu.CompilerParams(needs_layout_passes=False),
        scratch_types=[
            pltpu.VMEM((NCHUNK, C), jnp.int32),
            pltpu.VMEM((NCHUNK, C), jnp.int32),
            pltpu.VMEM((NCHUNK, C), jnp.float32),
            pltpu.VMEM((C, D), jnp.float32),
            pltpu.VMEM_SHARED((NP, D), jnp.float32),
            pltpu.SemaphoreType.DMA,
        ],
        name="sc_hop",
    )


def _sc_hop(h, row3, col3, norm3):
    return _sc_hop_kernel()(h, row3, col3, norm3)


def _tc_hop1_body(p_ref, x_ref, w0_ref, w1_ref, h_ref, acc_ref):
    hk = p_ref[0] + p_ref[1]
    h_ref[...] = hk
    acc_ref[...] = _bdot(x_ref[...], w0_ref[...]) + _bdot(hk, w1_ref[...])


def _tc_hop1(p, x, w0, w1):
    return pl.pallas_call(
        _tc_hop1_body,
        grid=(NBLK,),
        in_specs=[
            pl.BlockSpec((NC, BN, D), lambda i: (0, i, 0)),
            pl.BlockSpec((BN, D), lambda i: (i, 0)),
            pl.BlockSpec((D, D), lambda i: (0, 0)),
            pl.BlockSpec((D, D), lambda i: (0, 0)),
        ],
        out_specs=[
            pl.BlockSpec((BN, D), lambda i: (i, 0)),
            pl.BlockSpec((BN, D), lambda i: (i, 0)),
        ],
        out_shape=[
            jax.ShapeDtypeStruct((N, D), jnp.float32),
            jax.ShapeDtypeStruct((N, D), jnp.float32),
        ],
        name="tc_hop1_combine",
    )(p, x, w0, w1)


def _tc_hop2_body(p_ref, a_ref, w_ref, h_ref, acc_ref):
    hk = p_ref[0] + p_ref[1]
    h_ref[...] = hk
    acc_ref[...] = a_ref[...] + _bdot(hk, w_ref[...])


def _tc_hop2(p, acc, w):
    return pl.pallas_call(
        _tc_hop2_body,
        grid=(NBLK,),
        in_specs=[
            pl.BlockSpec((NC, BN, D), lambda i: (0, i, 0)),
            pl.BlockSpec((BN, D), lambda i: (i, 0)),
            pl.BlockSpec((D, D), lambda i: (0, 0)),
        ],
        out_specs=[
            pl.BlockSpec((BN, D), lambda i: (i, 0)),
            pl.BlockSpec((BN, D), lambda i: (i, 0)),
        ],
        out_shape=[
            jax.ShapeDtypeStruct((N, D), jnp.float32),
            jax.ShapeDtypeStruct((N, D), jnp.float32),
        ],
        name="tc_hop2_combine",
    )(p, acc, w)


def _tc_hop3_elu_body(p_ref, a_ref, w_ref, b_ref, y_ref, st_ref):
    i = pl.program_id(0)
    hk = p_ref[0] + p_ref[1]
    v = a_ref[...] + _bdot(hk, w_ref[...]) + b_ref[...]
    y = jnp.where(v > 0, v, jnp.exp(v) - 1.0)
    y_ref[...] = y
    s1 = jnp.sum(y, axis=0, keepdims=True)
    s2 = jnp.sum(y * y, axis=0, keepdims=True)
    s = jnp.concatenate([s1, s2], axis=0)

    @pl.when(i == 0)
    def _():
        st_ref[...] = s

    @pl.when(i > 0)
    def _():
        st_ref[...] = st_ref[...] + s


def _tc_hop3_elu(p, acc, w, b):
    return pl.pallas_call(
        _tc_hop3_elu_body,
        grid=(NBLK,),
        in_specs=[
            pl.BlockSpec((NC, BN, D), lambda i: (0, i, 0)),
            pl.BlockSpec((BN, D), lambda i: (i, 0)),
            pl.BlockSpec((D, D), lambda i: (0, 0)),
            pl.BlockSpec((1, D), lambda i: (0, 0)),
        ],
        out_specs=[
            pl.BlockSpec((BN, D), lambda i: (i, 0)),
            pl.BlockSpec((2, D), lambda i: (0, 0)),
        ],
        out_shape=[
            jax.ShapeDtypeStruct((N, D), jnp.float32),
            jax.ShapeDtypeStruct((2, D), jnp.float32),
        ],
        name="tc_hop3_elu_stats",
    )(p, acc, w, b)


def _tc_hop3_final_body(p_ref, a_ref, w_ref, b_ref, o_ref):
    hk = p_ref[0] + p_ref[1]
    o_ref[...] = a_ref[...] + _bdot(hk, w_ref[...]) + b_ref[...]


def _tc_hop3_final(p, acc, w, b):
    return pl.pallas_call(
        _tc_hop3_final_body,
        grid=(NBLK,),
        in_specs=[
            pl.BlockSpec((NC, BN, D), lambda i: (0, i, 0)),
            pl.BlockSpec((BN, D), lambda i: (i, 0)),
            pl.BlockSpec((D, D), lambda i: (0, 0)),
            pl.BlockSpec((1, D), lambda i: (0, 0)),
        ],
        out_specs=pl.BlockSpec((BN, D), lambda i: (i, 0)),
        out_shape=jax.ShapeDtypeStruct((N, D), jnp.float32),
        name="tc_hop3_final",
    )(p, acc, w, b)


def _tc_pool_body(y_ref, st_ref, gw_ref, gb_ref, gms_ref, kf_ref,
                  hh_ref, dpad_ref, m_ref, cb_ref, z_ref, s_ref):
    m = st_ref[0:1, :] * (1.0 / N)
    ey2 = st_ref[1:2, :] * (1.0 / N)
    ms = gms_ref[...]
    var = ey2 - (2.0 * ms - ms * ms) * m * m
    rstd = lax.rsqrt(var + 1e-5)
    z = gw_ref[...] * (y_ref[...] - m * ms) * rstd + gb_ref[...]
    z_ref[...] = z

    kf = kf_ref[...]
    cols = []
    for c in range(HEADS * CLUSTERS):
        diff = z - kf[c:c + 1, :]
        cols.append(jnp.sum(diff * diff, axis=1, keepdims=True))
    cols.append(jnp.full((z.shape[0], HK - HEADS * CLUSTERS), 1e30,
                         jnp.float32))
    d2 = jnp.concatenate(cols, axis=1)
    dist = 1.0 / (1.0 + d2)
    denom = jnp.dot(dist, hh_ref[...], precision=_prec,
                    preferred_element_type=jnp.float32) + dpad_ref[...]
    sn = dist / denom
    spre = _bdot(sn, m_ref[...]) + cb_ref[...]
    mx = jnp.max(spre, axis=1, keepdims=True)
    e = jnp.exp(spre - mx)
    s_ref[...] = e / jnp.sum(e, axis=1, keepdims=True)


def _tc_pool(y, st, gw, gb, gms, kf, hh, dpad, m_mat, cb):
    return pl.pallas_call(
        _tc_pool_body,
        grid=(NBLK,),
        in_specs=[
            pl.BlockSpec((BN, D), lambda i: (i, 0)),
            pl.BlockSpec((2, D), lambda i: (0, 0)),
            pl.BlockSpec((1, D), lambda i: (0, 0)),
            pl.BlockSpec((1, D), lambda i: (0, 0)),
            pl.BlockSpec((1, D), lambda i: (0, 0)),
            pl.BlockSpec((HK, D), lambda i: (0, 0)),
            pl.BlockSpec((HK, HK), lambda i: (0, 0)),
            pl.BlockSpec((1, HK), lambda i: (0, 0)),
            pl.BlockSpec((HK, CP), lambda i: (0, 0)),
            pl.BlockSpec((1, CP), lambda i: (0, 0)),
        ],
        out_specs=[
            pl.BlockSpec((BN, D), lambda i: (i, 0)),
            pl.BlockSpec((BN, CP), lambda i: (i, 0)),
        ],
        out_shape=[
            jax.ShapeDtypeStruct((N, D), jnp.float32),
            jax.ShapeDtypeStruct((N, CP), jnp.float32),
        ],
        name="tc_graphnorm_mempool",
    )(y, st, gw, gb, gms, kf, hh, dpad, m_mat, cb)


def _tc_xp_body(s_ref, z_ref, lin_ref, xp_ref):
    p = lax.dot_general(s_ref[...].astype(jnp.bfloat16),
                        z_ref[...].astype(jnp.bfloat16),
                        (((0,), (0,)), ((), ())),
                        preferred_element_type=jnp.float32)
    xp_ref[...] = _bdot(p, lin_ref[...])


def _tc_xp(s, z, lin):
    return pl.pallas_call(
        _tc_xp_body,
        grid=(1,),
        in_specs=[
            pl.BlockSpec((N, CP), lambda i: (0, 0)),
            pl.BlockSpec((N, D), lambda i: (0, 0)),
            pl.BlockSpec((D, D), lambda i: (0, 0)),
        ],
        out_specs=pl.BlockSpec((CP, D), lambda i: (0, 0)),
        out_shape=jax.ShapeDtypeStruct((CP, D), jnp.float32),
        name="tc_pool_xp",
    )(s, z, lin)


def _tc_unpool_body(s_ref, xp_ref, o_ref):
    o_ref[...] = _bdot(s_ref[...], xp_ref[...])


def _tc_unpool(s, xp):
    return pl.pallas_call(
        _tc_unpool_body,
        grid=(NBLK,),
        in_specs=[
            pl.BlockSpec((BN, CP), lambda i: (i, 0)),
            pl.BlockSpec((CP, D), lambda i: (0, 0)),
        ],
        out_specs=pl.BlockSpec((BN, D), lambda i: (i, 0)),
        out_shape=jax.ShapeDtypeStruct((N, D), jnp.float32),
        name="tc_unpool",
    )(s, xp)


def _pool_consts():
    hsel = np.zeros((HK, HEADS), np.float32)
    csel = np.zeros((HK, CP), np.float32)
    for h in range(HEADS):
        for k in range(CLUSTERS):
            c = h * CLUSTERS + k
            hsel[c, h] = 1.0
            csel[c, k] = 1.0
    hh = hsel @ hsel.T
    dpad = np.zeros((1, HK), np.float32)
    dpad[0, HEADS * CLUSTERS:] = 1.0
    cb = np.zeros((1, CP), np.float32)
    cb[0, CLUSTERS:] = -1e30
    return jnp.asarray(hh), jnp.asarray(dpad), jnp.asarray(csel), jnp.asarray(cb)


def _prep_pool_args(keys, headw, csel):
    kflat = keys.reshape(HEADS * CLUSTERS, D)
    kf = jnp.zeros((HK, D), jnp.float32).at[:HEADS * CLUSTERS, :].set(kflat)
    hw = jnp.zeros((HK, 1), jnp.float32).at[:HEADS * CLUSTERS, 0].set(
        jnp.repeat(headw, CLUSTERS))
    return kf, csel * hw


def kernel(x, edge_index, edge_weight, W1, b1, W2, b2, W3, b3,
           gn1_w, gn1_b, gn1_ms, gn2_w, gn2_b, gn2_ms,
           m1_keys, m1_headw, m1_lin, m2_keys, m2_headw, m2_lin):
    pad = EP - E
    row = jnp.concatenate([edge_index[0], jnp.zeros((pad,), jnp.int32)])
    col = jnp.concatenate([edge_index[1], jnp.zeros((pad,), jnp.int32)])
    w = jnp.concatenate([edge_weight, jnp.zeros((pad,), jnp.float32)])
    row3 = row.reshape(NW, NCHUNK, C)
    col3 = col.reshape(NW, NCHUNK, C)
    w3 = w.reshape(NW, NCHUNK, C)

    norm3 = _sc_norm(row3, col3, w3)

    hh, dpad, csel, cb = _pool_consts()
    b1r = b1.reshape(1, D)
    b2r = b2.reshape(1, D)
    b3r = b3.reshape(1, D)
    gn1 = (gn1_w.reshape(1, D), gn1_b.reshape(1, D), gn1_ms.reshape(1, D))
    gn2 = (gn2_w.reshape(1, D), gn2_b.reshape(1, D), gn2_ms.reshape(1, D))
    kf1, mm1 = _prep_pool_args(m1_keys, m1_headw, csel)
    kf2, mm2 = _prep_pool_args(m2_keys, m2_headw, csel)

    def tag_layer(h, W, br, final):
        p = _sc_hop(h, row3, col3, norm3)
        h1, acc = _tc_hop1(p, h, W[0], W[1])
        p = _sc_hop(h1, row3, col3, norm3)
        h2, acc = _tc_hop2(p, acc, W[2])
        p = _sc_hop(h2, row3, col3, norm3)
        if final:
            return _tc_hop3_final(p, acc, W[3], br)
        return _tc_hop3_elu(p, acc, W[3], br)

    y, st = tag_layer(x, W1, b1r, final=False)
    z, s = _tc_pool(y, st, *gn1, kf1, hh, dpad, mm1, cb)
    xp = _tc_xp(s, z, m1_lin)
    h = _tc_unpool(s, xp)

    y, st = tag_layer(h, W2, b2r, final=False)
    z, s = _tc_pool(y, st, *gn2, kf2, hh, dpad, mm2, cb)
    xp = _tc_xp(s, z, m2_lin)
    h = _tc_unpool(s, xp)

    return tag_layer(h, W3, b3r, final=True)


# speedup vs baseline: 3.7194x; 1.1791x over previous
import functools

import numpy as np
import jax
import jax.numpy as jnp
from jax import lax
from jax.experimental import pallas as pl
from jax.experimental.pallas import tpu as pltpu
from jax.experimental.pallas import tpu_sc as plsc

N = 10000
D = 128
E = 160000
K_HOPS = 3
HEADS = 3
CLUSTERS = 3

NC = 2
NS = 16
NW = NC * NS
ET = 5120
EP = NW * ET
C = 128
NCHUNK = ET // C
RPT = N // NS
NP = 10240
RPTP = NP // NS

BN = 400
NBLK = N // BN
HK = 16
CP = 8

@functools.cache
def _sc_mesh():
    return plsc.VectorSubcoreMesh(core_axis_name="c", subcore_axis_name="s",
                                  num_cores=NC, num_subcores=NS)

_prec = lax.Precision.HIGHEST


def _bdot(a, b):
    return jnp.dot(a.astype(jnp.bfloat16), b.astype(jnp.bfloat16),
                   preferred_element_type=jnp.float32)


def _f32(x):
    return jnp.full((16,), x, dtype=jnp.float32)


def _sc_norm_body(row_h, col_h, w_h, norm_h,
                  rowv, colv, wv, normv, degl, dsl, cbuf, dinvl, stage, dinvs):
    cid = lax.axis_index("c")
    sid = lax.axis_index("s")
    wid = sid * NC + cid

    def _z(i, _):
        degl[pl.ds(i * 16, 16)] = jnp.zeros((16,), jnp.float32)
        return 0
    lax.fori_loop(0, NP // 16, _z, 0)

    for r in range(2):
        b = sid * 2 + r
        pltpu.sync_copy(col_h.at[b], colv)
        pltpu.sync_copy(w_h.at[b], wv)
        for j in range(NCHUNK):
            def _deg(k, _, j=j):
                sl = pl.ds(k * 16, 16)
                plsc.addupdate_scatter(degl, [colv[j, sl]], wv[j, sl])
                return 0
            lax.fori_loop(0, C // 16, _deg, 0)

    pltpu.sync_copy(degl, stage.at[sid])
    plsc.subcore_barrier()
    pltpu.sync_copy(stage.at[:, pl.ds(sid * RPTP, RPTP)], cbuf)

    def _sum(k, _):
        sl = pl.ds(k * 16, 16)
        v = cbuf[0, sl]
        for t in range(1, NS):
            v = v + cbuf[t, sl]
        dsl[sl] = v
        return 0
    lax.fori_loop(0, RPTP // 16, _sum, 0)

    def _dinv(k, _):
        sl = pl.ds(k * 16, 16)
        d = dsl[sl]
        i = plsc.bitcast(d, jnp.int32)
        i = jnp.full((16,), 0x5F3759DF, jnp.int32) - lax.shift_right_logical(
            i, jnp.full((16,), 1, jnp.int32))
        y = plsc.bitcast(i, jnp.float32)
        for _n in range(3):
            y = y * (_f32(1.5) - _f32(0.5) * d * y * y)
        dsl[sl] = jnp.where(d > jnp.zeros((16,), jnp.float32), y,
                            jnp.zeros((16,), jnp.float32))
        return 0
    lax.fori_loop(0, RPTP // 16, _dinv, 0)
    pltpu.sync_copy(dsl, dinvs.at[pl.ds(sid * RPTP, RPTP)])
    plsc.subcore_barrier()

    pltpu.sync_copy(dinvs, dinvl)
    pltpu.sync_copy(row_h.at[wid], rowv)
    pltpu.sync_copy(col_h.at[wid], colv)
    pltpu.sync_copy(w_h.at[wid], wv)
    for j in range(NCHUNK):
        def _norm(k, _, j=j):
            sl = pl.ds(k * 16, 16)
            dr = plsc.load_gather(dinvl, [rowv[j, sl]])
            dc = plsc.load_gather(dinvl, [colv[j, sl]])
            normv[j, sl] = dr * wv[j, sl] * dc
            return 0
        lax.fori_loop(0, C // 16, _norm, 0)
    pltpu.sync_copy(normv, norm_h.at[wid])


@functools.cache
def _sc_norm_kernel():
    return pl.kernel(
        _sc_norm_body,
        out_type=jax.ShapeDtypeStruct((NW, NCHUNK, C), jnp.float32),
        mesh=_sc_mesh(),
        compiler_params=pltpu.CompilerParams(needs_layout_passes=False),
        scratch_types=[
        pltpu.VMEM((NCHUNK, C), jnp.int32),
        pltpu.VMEM((NCHUNK, C), jnp.int32),
        pltpu.VMEM((NCHUNK, C), jnp.float32),
        pltpu.VMEM((NCHUNK, C), jnp.float32),
        pltpu.VMEM((NP,), jnp.float32),
        pltpu.VMEM((RPTP,), jnp.float32),
        pltpu.VMEM((NS, RPTP), jnp.float32),
        pltpu.VMEM((NP,), jnp.float32),
        pltpu.VMEM_SHARED((NS, NP), jnp.float32),
        pltpu.VMEM_SHARED((NP,), jnp.float32),
        ],
        name="sc_edge_norm",
    )


def _sc_norm(row3, col3, w3):
    return _sc_norm_kernel()(row3, col3, w3)


def _sc_hop_body(h_h, row_h, col_h, norm_h, part_h,
                 rowv, colv, normv, gbuf0, gbuf1, acc, sem0, sem1):
    cid = lax.axis_index("c")
    sid = lax.axis_index("s")
    wid = sid * NC + cid

    pltpu.sync_copy(row_h.at[wid], rowv)
    pltpu.sync_copy(col_h.at[wid], colv)
    pltpu.sync_copy(norm_h.at[wid], normv)

    def _zg(i, _):
        for k in range(D // 16):
            gbuf0[i, pl.ds(k * 16, 16)] = jnp.zeros((16,), jnp.float32)
        return 0
    lax.fori_loop(0, C, _zg, 0)
    for t in range(RPTP // C):
        pltpu.sync_copy(gbuf0, acc.at[pl.ds(sid * RPTP + t * C, C)])
    plsc.subcore_barrier()

    def _work(j, gbuf):
        def _scale(g, _):
            nv = normv[j, pl.ds(g * 16, 16)]
            for l in range(16):
                s = nv[l]
                e = g * 16 + l
                for k in range(D // 16):
                    sl = pl.ds(k * 16, 16)
                    gbuf[e, sl] = gbuf[e, sl] * s
            return 0
        lax.fori_loop(0, C // 16, _scale, 0)
        pltpu.sync_copy(gbuf, acc.at[colv.at[j]], add=True)

    pltpu.async_copy(h_h.at[rowv.at[0]], gbuf0, sem0)

    def _pair(jj, _):
        j0 = 2 * jj
        j1 = 2 * jj + 1
        pltpu.async_copy(h_h.at[rowv.at[j1]], gbuf1, sem1)
        pltpu.make_async_copy(h_h.at[rowv.at[j0]], gbuf0, sem0).wait()
        _work(j0, gbuf0)
        jn = jnp.minimum(j0 + 2, NCHUNK - 1)
        pltpu.async_copy(h_h.at[rowv.at[jn]], gbuf0, sem0)
        pltpu.make_async_copy(h_h.at[rowv.at[j1]], gbuf1, sem1).wait()
        _work(j1, gbuf1)
        return 0
    lax.fori_loop(0, NCHUNK // 2, _pair, 0)
    pltpu.make_async_copy(h_h.at[rowv.at[NCHUNK - 1]], gbuf0, sem0).wait()
    plsc.subcore_barrier()

    pltpu.sync_copy(acc.at[pl.ds(sid * RPTP, RPTP)],
                    part_h.at[cid, pl.ds(sid * RPTP, RPTP)])


@functools.cache
def _sc_hop_kernel():
    return pl.kernel(
        _sc_hop_body,
        out_type=jax.ShapeDtypeStruct((NC, NP, D), jnp.float32),
        mesh=_sc_mesh(),
        compiler_params=pltpu.CompilerParams(needs_layout_passes=False),
        scratch_types=[
            pltpu.VMEM((NCHUNK, C), jnp.int32),
            pltpu.VMEM((NCHUNK, C), jnp.int32),
            pltpu.VMEM((NCHUNK, C), jnp.float32),
            pltpu.VMEM((C, D), jnp.float32),
            pltpu.VMEM((C, D), jnp.float32),
            pltpu.VMEM_SHARED((NP, D), jnp.float32),
            pltpu.SemaphoreType.DMA,
            pltpu.SemaphoreType.DMA,
        ],
        name="sc_hop",
    )


def _sc_hop(h, row3, col3, norm3):
    return _sc_hop_kernel()(h, row3, col3, norm3)


def _tc_hop1_body(p_ref, x_ref, w0_ref, w1_ref, h_ref, acc_ref):
    hk = p_ref[0] + p_ref[1]
    h_ref[...] = hk
    acc_ref[...] = _bdot(x_ref[...], w0_ref[...]) + _bdot(hk, w1_ref[...])


def _tc_hop1(p, x, w0, w1):
    return pl.pallas_call(
        _tc_hop1_body,
        grid=(NBLK,),
        in_specs=[
            pl.BlockSpec((NC, BN, D), lambda i: (0, i, 0)),
            pl.BlockSpec((BN, D), lambda i: (i, 0)),
            pl.BlockSpec((D, D), lambda i: (0, 0)),
            pl.BlockSpec((D, D), lambda i: (0, 0)),
        ],
        out_specs=[
            pl.BlockSpec((BN, D), lambda i: (i, 0)),
            pl.BlockSpec((BN, D), lambda i: (i, 0)),
        ],
        out_shape=[
            jax.ShapeDtypeStruct((N, D), jnp.float32),
            jax.ShapeDtypeStruct((N, D), jnp.float32),
        ],
        name="tc_hop1_combine",
    )(p, x, w0, w1)


def _tc_hop2_body(p_ref, a_ref, w_ref, h_ref, acc_ref):
    hk = p_ref[0] + p_ref[1]
    h_ref[...] = hk
    acc_ref[...] = a_ref[...] + _bdot(hk, w_ref[...])


def _tc_hop2(p, acc, w):
    return pl.pallas_call(
        _tc_hop2_body,
        grid=(NBLK,),
        in_specs=[
            pl.BlockSpec((NC, BN, D), lambda i: (0, i, 0)),
            pl.BlockSpec((BN, D), lambda i: (i, 0)),
            pl.BlockSpec((D, D), lambda i: (0, 0)),
        ],
        out_specs=[
            pl.BlockSpec((BN, D), lambda i: (i, 0)),
            pl.BlockSpec((BN, D), lambda i: (i, 0)),
        ],
        out_shape=[
            jax.ShapeDtypeStruct((N, D), jnp.float32),
            jax.ShapeDtypeStruct((N, D), jnp.float32),
        ],
        name="tc_hop2_combine",
    )(p, acc, w)


def _tc_hop3_elu_body(p_ref, a_ref, w_ref, b_ref, y_ref, st_ref):
    i = pl.program_id(0)
    hk = p_ref[0] + p_ref[1]
    v = a_ref[...] + _bdot(hk, w_ref[...]) + b_ref[...]
    y = jnp.where(v > 0, v, jnp.exp(v) - 1.0)
    y_ref[...] = y
    s1 = jnp.sum(y, axis=0, keepdims=True)
    s2 = jnp.sum(y * y, axis=0, keepdims=True)
    s = jnp.concatenate([s1, s2], axis=0)

    @pl.when(i == 0)
    def _():
        st_ref[...] = s

    @pl.when(i > 0)
    def _():
        st_ref[...] = st_ref[...] + s


def _tc_hop3_elu(p, acc, w, b):
    return pl.pallas_call(
        _tc_hop3_elu_body,
        grid=(NBLK,),
        in_specs=[
            pl.BlockSpec((NC, BN, D), lambda i: (0, i, 0)),
            pl.BlockSpec((BN, D), lambda i: (i, 0)),
            pl.BlockSpec((D, D), lambda i: (0, 0)),
            pl.BlockSpec((1, D), lambda i: (0, 0)),
        ],
        out_specs=[
            pl.BlockSpec((BN, D), lambda i: (i, 0)),
            pl.BlockSpec((2, D), lambda i: (0, 0)),
        ],
        out_shape=[
            jax.ShapeDtypeStruct((N, D), jnp.float32),
            jax.ShapeDtypeStruct((2, D), jnp.float32),
        ],
        name="tc_hop3_elu_stats",
    )(p, acc, w, b)


def _tc_hop3_final_body(p_ref, a_ref, w_ref, b_ref, o_ref):
    hk = p_ref[0] + p_ref[1]
    o_ref[...] = a_ref[...] + _bdot(hk, w_ref[...]) + b_ref[...]


def _tc_hop3_final(p, acc, w, b):
    return pl.pallas_call(
        _tc_hop3_final_body,
        grid=(NBLK,),
        in_specs=[
            pl.BlockSpec((NC, BN, D), lambda i: (0, i, 0)),
            pl.BlockSpec((BN, D), lambda i: (i, 0)),
            pl.BlockSpec((D, D), lambda i: (0, 0)),
            pl.BlockSpec((1, D), lambda i: (0, 0)),
        ],
        out_specs=pl.BlockSpec((BN, D), lambda i: (i, 0)),
        out_shape=jax.ShapeDtypeStruct((N, D), jnp.float32),
        name="tc_hop3_final",
    )(p, acc, w, b)


def _tc_pool_body(y_ref, st_ref, gw_ref, gb_ref, gms_ref, kf_ref,
                  hh_ref, dpad_ref, m_ref, cb_ref, z_ref, s_ref):
    m = st_ref[0:1, :] * (1.0 / N)
    ey2 = st_ref[1:2, :] * (1.0 / N)
    ms = gms_ref[...]
    var = ey2 - (2.0 * ms - ms * ms) * m * m
    rstd = lax.rsqrt(var + 1e-5)
    z = gw_ref[...] * (y_ref[...] - m * ms) * rstd + gb_ref[...]
    z_ref[...] = z

    kf = kf_ref[...]
    cols = []
    for c in range(HEADS * CLUSTERS):
        diff = z - kf[c:c + 1, :]
        cols.append(jnp.sum(diff * diff, axis=1, keepdims=True))
    cols.append(jnp.full((z.shape[0], HK - HEADS * CLUSTERS), 1e30,
                         jnp.float32))
    d2 = jnp.concatenate(cols, axis=1)
    dist = 1.0 / (1.0 + d2)
    denom = jnp.dot(dist, hh_ref[...], precision=_prec,
                    preferred_element_type=jnp.float32) + dpad_ref[...]
    sn = dist / denom
    spre = _bdot(sn, m_ref[...]) + cb_ref[...]
    mx = jnp.max(spre, axis=1, keepdims=True)
    e = jnp.exp(spre - mx)
    s_ref[...] = e / jnp.sum(e, axis=1, keepdims=True)


def _tc_pool(y, st, gw, gb, gms, kf, hh, dpad, m_mat, cb):
    return pl.pallas_call(
        _tc_pool_body,
        grid=(NBLK,),
        in_specs=[
            pl.BlockSpec((BN, D), lambda i: (i, 0)),
            pl.BlockSpec((2, D), lambda i: (0, 0)),
            pl.BlockSpec((1, D), lambda i: (0, 0)),
            pl.BlockSpec((1, D), lambda i: (0, 0)),
            pl.BlockSpec((1, D), lambda i: (0, 0)),
            pl.BlockSpec((HK, D), lambda i: (0, 0)),
            pl.BlockSpec((HK, HK), lambda i: (0, 0)),
            pl.BlockSpec((1, HK), lambda i: (0, 0)),
            pl.BlockSpec((HK, CP), lambda i: (0, 0)),
            pl.BlockSpec((1, CP), lambda i: (0, 0)),
        ],
        out_specs=[
            pl.BlockSpec((BN, D), lambda i: (i, 0)),
            pl.BlockSpec((BN, CP), lambda i: (i, 0)),
        ],
        out_shape=[
            jax.ShapeDtypeStruct((N, D), jnp.float32),
            jax.ShapeDtypeStruct((N, CP), jnp.float32),
        ],
        name="tc_graphnorm_mempool",
    )(y, st, gw, gb, gms, kf, hh, dpad, m_mat, cb)


def _tc_xp_body(s_ref, z_ref, lin_ref, xp_ref):
    p = lax.dot_general(s_ref[...].astype(jnp.bfloat16),
                        z_ref[...].astype(jnp.bfloat16),
                        (((0,), (0,)), ((), ())),
                        preferred_element_type=jnp.float32)
    xp_ref[...] = _bdot(p, lin_ref[...])


def _tc_xp(s, z, lin):
    return pl.pallas_call(
        _tc_xp_body,
        grid=(1,),
        in_specs=[
            pl.BlockSpec((N, CP), lambda i: (0, 0)),
            pl.BlockSpec((N, D), lambda i: (0, 0)),
            pl.BlockSpec((D, D), lambda i: (0, 0)),
        ],
        out_specs=pl.BlockSpec((CP, D), lambda i: (0, 0)),
        out_shape=jax.ShapeDtypeStruct((CP, D), jnp.float32),
        name="tc_pool_xp",
    )(s, z, lin)


def _tc_unpool_body(s_ref, xp_ref, o_ref):
    o_ref[...] = _bdot(s_ref[...], xp_ref[...])


def _tc_unpool(s, xp):
    return pl.pallas_call(
        _tc_unpool_body,
        grid=(NBLK,),
        in_specs=[
            pl.BlockSpec((BN, CP), lambda i: (i, 0)),
            pl.BlockSpec((CP, D), lambda i: (0, 0)),
        ],
        out_specs=pl.BlockSpec((BN, D), lambda i: (i, 0)),
        out_shape=jax.ShapeDtypeStruct((N, D), jnp.float32),
        name="tc_unpool",
    )(s, xp)


def _pool_consts():
    hsel = np.zeros((HK, HEADS), np.float32)
    csel = np.zeros((HK, CP), np.float32)
    for h in range(HEADS):
        for k in range(CLUSTERS):
            c = h * CLUSTERS + k
            hsel[c, h] = 1.0
            csel[c, k] = 1.0
    hh = hsel @ hsel.T
    dpad = np.zeros((1, HK), np.float32)
    dpad[0, HEADS * CLUSTERS:] = 1.0
    cb = np.zeros((1, CP), np.float32)
    cb[0, CLUSTERS:] = -1e30
    return jnp.asarray(hh), jnp.asarray(dpad), jnp.asarray(csel), jnp.asarray(cb)


def _prep_pool_args(keys, headw, csel):
    kflat = keys.reshape(HEADS * CLUSTERS, D)
    kf = jnp.zeros((HK, D), jnp.float32).at[:HEADS * CLUSTERS, :].set(kflat)
    hw = jnp.zeros((HK, 1), jnp.float32).at[:HEADS * CLUSTERS, 0].set(
        jnp.repeat(headw, CLUSTERS))
    return kf, csel * hw


def kernel(x, edge_index, edge_weight, W1, b1, W2, b2, W3, b3,
           gn1_w, gn1_b, gn1_ms, gn2_w, gn2_b, gn2_ms,
           m1_keys, m1_headw, m1_lin, m2_keys, m2_headw, m2_lin):
    pad = EP - E
    row = jnp.concatenate([edge_index[0], jnp.zeros((pad,), jnp.int32)])
    col = jnp.concatenate([edge_index[1], jnp.zeros((pad,), jnp.int32)])
    w = jnp.concatenate([edge_weight, jnp.zeros((pad,), jnp.float32)])
    row3 = row.reshape(NW, NCHUNK, C)
    col3 = col.reshape(NW, NCHUNK, C)
    w3 = w.reshape(NW, NCHUNK, C)

    norm3 = _sc_norm(row3, col3, w3)

    hh, dpad, csel, cb = _pool_consts()
    b1r = b1.reshape(1, D)
    b2r = b2.reshape(1, D)
    b3r = b3.reshape(1, D)
    gn1 = (gn1_w.reshape(1, D), gn1_b.reshape(1, D), gn1_ms.reshape(1, D))
    gn2 = (gn2_w.reshape(1, D), gn2_b.reshape(1, D), gn2_ms.reshape(1, D))
    kf1, mm1 = _prep_pool_args(m1_keys, m1_headw, csel)
    kf2, mm2 = _prep_pool_args(m2_keys, m2_headw, csel)

    def tag_layer(h, W, br, final):
        p = _sc_hop(h, row3, col3, norm3)
        h1, acc = _tc_hop1(p, h, W[0], W[1])
        p = _sc_hop(h1, row3, col3, norm3)
        h2, acc = _tc_hop2(p, acc, W[2])
        p = _sc_hop(h2, row3, col3, norm3)
        if final:
            return _tc_hop3_final(p, acc, W[3], br)
        return _tc_hop3_elu(p, acc, W[3], br)

    y, st = tag_layer(x, W1, b1r, final=False)
    z, s = _tc_pool(y, st, *gn1, kf1, hh, dpad, mm1, cb)
    xp = _tc_xp(s, z, m1_lin)
    h = _tc_unpool(s, xp)

    y, st = tag_layer(h, W2, b2r, final=False)
    z, s = _tc_pool(y, st, *gn2, kf2, hh, dpad, mm2, cb)
    xp = _tc_xp(s, z, m2_lin)
    h = _tc_unpool(s, xp)

    return tag_layer(h, W3, b3r, final=True)
